# baseline (speedup 1.0000x reference)
import functools

import jax
import jax.numpy as jnp
from jax import lax
from jax.experimental import pallas as pl
from jax.experimental.pallas import tpu as pltpu
from jax.experimental.pallas import tpu_sc as plsc

_T = 16
_LANES = 16
_NC, _NS = 2, 16
_NW = _NC * _NS
_DESIRED = 5.0
_UNDESIRED = 0.0


@functools.partial(jax.jit, static_argnames=("nb", "nn", "rows"))
def _sc_partials(out3d, tgt2d, *, nb, nn, rows):
    rows_per_w = nb // _NW
    nchunk = rows_per_w // rows
    k = rows * nn

    mesh = plsc.VectorSubcoreMesh(core_axis_name="c", subcore_axis_name="s")

    @functools.partial(
        pl.kernel,
        mesh=mesh,
        out_type=jax.ShapeDtypeStruct((_NW, _LANES), jnp.float32),
        scratch_types=[
            pltpu.VMEM((_T, rows, nn), jnp.float32),
            pltpu.VMEM((rows, nn), jnp.int32),
            pltpu.VMEM((_LANES,), jnp.float32),
            pltpu.SemaphoreType.DMA,
            pltpu.SemaphoreType.DMA,
        ],
    )
    def partials(out_hbm, tgt_hbm, res_hbm, buf, tbuf, accv, sem, tsem):
        wid = lax.axis_index("c") * _NS + lax.axis_index("s")
        base = wid * rows_per_w

        def chunk_body(ci, acc):
            off = base + ci * rows
            cp = pltpu.async_copy(out_hbm.at[:, pl.ds(off, rows), :], buf, sem)
            tcp = pltpu.async_copy(tgt_hbm.at[pl.ds(off, rows), :], tbuf, tsem)
            cp.wait()
            tcp.wait()

            def j_body(j, acc):
                r = j // (nn // _LANES)
                col = (j % (nn // _LANES)) * _LANES
                cnt = buf[0, r, pl.ds(col, _LANES)]
                for t in range(1, _T):
                    cnt = cnt + buf[t, r, pl.ds(col, _LANES)]
                tg = tbuf[r, pl.ds(col, _LANES)].astype(jnp.float32)
                delta = (cnt - tg) * (1.0 / _T)
                zero = jnp.zeros_like(delta)
                m = ((tg == _DESIRED) & (delta > zero)) | (
                    (tg == _UNDESIRED) & (delta < zero))
                delta = jnp.where(m, zero, delta)
                return acc + delta * delta

            return lax.fori_loop(0, k // _LANES, j_body, acc)

        acc = lax.fori_loop(0, nchunk, chunk_body,
                            jnp.zeros((_LANES,), jnp.float32))
        accv[...] = acc
        pltpu.sync_copy(accv, res_hbm.at[wid])

    return partials(out3d, tgt2d)


def kernel(output, target):
    t, nb, nn = output.shape
    parts = _sc_partials(output, target.astype(jnp.int32),
                         nb=nb, nn=nn, rows=4)
    return 0.5 * t * jnp.sum(parts)


# speedup vs baseline: 1.4547x; 1.4547x over previous
import functools

import jax
import jax.numpy as jnp
from jax import lax
from jax.experimental import pallas as pl
from jax.experimental.pallas import tpu as pltpu
from jax.experimental.pallas import tpu_sc as plsc

_T = 16
_LANES = 16
_NC, _NS = 2, 16
_NW = _NC * _NS
_DESIRED = 5.0
_UNDESIRED = 0.0


@functools.partial(jax.jit, static_argnames=("nb", "nn", "rows"))
def _sc_partials(out3d, tgt2d, *, nb, nn, rows):
    rows_per_w = nb // _NW
    nchunk = rows_per_w // rows

    mesh = plsc.VectorSubcoreMesh(core_axis_name="c", subcore_axis_name="s")

    def compute_chunk(buf, tbuf, acc):
        def r_body(r, acc):
            def j_body(j, acc):
                col = j * _LANES
                cnt = buf[0, r, pl.ds(col, _LANES)]
                for t in range(1, _T):
                    cnt = cnt + buf[t, r, pl.ds(col, _LANES)]
                tg = tbuf[r, pl.ds(col, _LANES)].astype(jnp.float32)
                delta = (cnt - tg) * (1.0 / _T)
                zero = jnp.zeros_like(delta)
                m = ((tg == _DESIRED) & (delta > zero)) | (
                    (tg == _UNDESIRED) & (delta < zero))
                delta = jnp.where(m, zero, delta)
                return acc + delta * delta

            return lax.fori_loop(0, nn // _LANES, j_body, acc)

        return lax.fori_loop(0, rows, r_body, acc)

    @functools.partial(
        pl.kernel,
        mesh=mesh,
        out_type=jax.ShapeDtypeStruct((_NW, _LANES), jnp.float32),
        scratch_types=[
            pltpu.VMEM((_T, rows, nn), jnp.float32),
            pltpu.VMEM((_T, rows, nn), jnp.float32),
            pltpu.VMEM((rows, nn), jnp.int32),
            pltpu.VMEM((rows, nn), jnp.int32),
            pltpu.VMEM((_LANES,), jnp.float32),
            pltpu.SemaphoreType.DMA,
            pltpu.SemaphoreType.DMA,
            pltpu.SemaphoreType.DMA,
            pltpu.SemaphoreType.DMA,
        ],
    )
    def partials(out_hbm, tgt_hbm, res_hbm, buf_a, buf_b, tbuf_a, tbuf_b,
                 accv, sem_a, sem_b, tsem_a, tsem_b):
        wid = lax.axis_index("c") * _NS + lax.axis_index("s")
        base = wid * rows_per_w

        def start(ci, buf, tbuf, sem, tsem):
            off = base + ci * rows
            pltpu.async_copy(out_hbm.at[:, pl.ds(off, rows), :], buf, sem)
            pltpu.async_copy(tgt_hbm.at[pl.ds(off, rows), :], tbuf, tsem)

        def wait(buf, tbuf, sem, tsem):
            pltpu.make_async_copy(
                out_hbm.at[:, pl.ds(base, rows), :], buf, sem).wait()
            pltpu.make_async_copy(
                tgt_hbm.at[pl.ds(base, rows), :], tbuf, tsem).wait()

        start(0, buf_a, tbuf_a, sem_a, tsem_a)

        def pair_body(i, acc):
            c = 2 * i
            start(c + 1, buf_b, tbuf_b, sem_b, tsem_b)
            wait(buf_a, tbuf_a, sem_a, tsem_a)
            acc = compute_chunk(buf_a, tbuf_a, acc)

            @pl.when(c + 2 < nchunk)
            def _():
                start(c + 2, buf_a, tbuf_a, sem_a, tsem_a)

            wait(buf_b, tbuf_b, sem_b, tsem_b)
            return compute_chunk(buf_b, tbuf_b, acc)

        acc = lax.fori_loop(0, nchunk // 2, pair_body,
                            jnp.zeros((_LANES,), jnp.float32))
        accv[...] = acc
        pltpu.sync_copy(accv, res_hbm.at[wid])

    return partials(out3d, tgt2d)


def kernel(output, target):
    t, nb, nn = output.shape
    parts = _sc_partials(output, target.astype(jnp.int32),
                         nb=nb, nn=nn, rows=4)
    return 0.5 * t * jnp.sum(parts)


# speedup vs baseline: 2.7236x; 1.8723x over previous
import functools

import jax
import jax.numpy as jnp
from jax import lax
from jax.experimental import pallas as pl
from jax.experimental.pallas import tpu as pltpu
from jax.experimental.pallas import tpu_sc as plsc

_T = 16
_LANES = 16
_NC, _NS = 2, 16
_NW = _NC * _NS
_DESIRED = 5.0
_UNDESIRED = 0.0


@functools.partial(jax.jit, static_argnames=("nb", "nn", "rows"))
def _sc_partials(out3d, tgt2d, *, nb, nn, rows):
    rows_per_w = nb // _NW
    nchunk = rows_per_w // rows

    mesh = plsc.VectorSubcoreMesh(core_axis_name="c", subcore_axis_name="s")

    def compute_chunk(buf, tbuf, acc):
        def r_body(r, acc):
            def j_body(j, acc):
                col = j * _LANES
                cnt = buf[0, r, pl.ds(col, _LANES)]
                for t in range(1, _T):
                    cnt = cnt + buf[t, r, pl.ds(col, _LANES)]
                tg = tbuf[r, pl.ds(col, _LANES)].astype(jnp.float32)
                delta = (cnt - tg) * (1.0 / _T)
                zero = jnp.zeros_like(delta)
                m = ((tg == _DESIRED) & (delta > zero)) | (
                    (tg == _UNDESIRED) & (delta < zero))
                delta = jnp.where(m, zero, delta)
                return acc + delta * delta

            return lax.fori_loop(0, nn // _LANES, j_body, acc)

        return lax.fori_loop(0, rows, r_body, acc)

    @functools.partial(
        pl.kernel,
        mesh=mesh,
        out_type=jax.ShapeDtypeStruct((_NW, _LANES), jnp.float32),
        scratch_types=[
            pltpu.VMEM((_T, rows, nn), jnp.float32),
            pltpu.VMEM((_T, rows, nn), jnp.float32),
            pltpu.VMEM((rows, nn), jnp.int32),
            pltpu.VMEM((rows, nn), jnp.int32),
            pltpu.VMEM((_LANES,), jnp.float32),
            pltpu.SemaphoreType.DMA,
            pltpu.SemaphoreType.DMA,
            pltpu.SemaphoreType.DMA,
            pltpu.SemaphoreType.DMA,
        ],
    )
    def partials(out_hbm, tgt_hbm, res_hbm, buf_a, buf_b, tbuf_a, tbuf_b,
                 accv, sem_a, sem_b, tsem_a, tsem_b):
        wid = lax.axis_index("c") * _NS + lax.axis_index("s")
        base = wid * rows_per_w

        def start(ci, buf, tbuf, sem, tsem):
            off = base + ci * rows
            pltpu.async_copy(out_hbm.at[:, pl.ds(off, rows), :], buf, sem)
            pltpu.async_copy(tgt_hbm.at[pl.ds(off, rows), :], tbuf, tsem)

        def wait(buf, tbuf, sem, tsem):
            pltpu.make_async_copy(
                out_hbm.at[:, pl.ds(base, rows), :], buf, sem).wait()
            pltpu.make_async_copy(
                tgt_hbm.at[pl.ds(base, rows), :], tbuf, tsem).wait()

        start(0, buf_a, tbuf_a, sem_a, tsem_a)

        def pair_body(i, acc):
            c = 2 * i
            start(c + 1, buf_b, tbuf_b, sem_b, tsem_b)
            wait(buf_a, tbuf_a, sem_a, tsem_a)
            acc = compute_chunk(buf_a, tbuf_a, acc)

            @pl.when(c + 2 < nchunk)
            def _():
                start(c + 2, buf_a, tbuf_a, sem_a, tsem_a)

            wait(buf_b, tbuf_b, sem_b, tsem_b)
            return compute_chunk(buf_b, tbuf_b, acc)

        acc = lax.fori_loop(0, nchunk // 2, pair_body,
                            jnp.zeros((_LANES,), jnp.float32))
        accv[...] = acc
        pltpu.sync_copy(accv, res_hbm.at[wid])

    return partials(out3d, tgt2d)


def _tc_loss_block(out_ref, tgt_ref, res_ref):
    i = pl.program_id(0)
    x = out_ref[...]
    cnt = jnp.sum(x, axis=0)
    tg = tgt_ref[...].astype(jnp.float32)
    delta = (cnt - tg) * (1.0 / _T)
    zero = jnp.zeros_like(delta)
    m = ((tg == _DESIRED) & (delta > zero)) | (
        (tg == _UNDESIRED) & (delta < zero))
    delta = jnp.where(m, zero, delta)
    part = jnp.sum(delta * delta)

    @pl.when(i == 0)
    def _():
        res_ref[...] = jnp.zeros_like(res_ref)

    res_ref[...] += part * jnp.ones((1, 128), jnp.float32)


@functools.partial(jax.jit, static_argnames=("rb",))
def _tc_partial(out3d, tgt2d, *, rb):
    t, nb, nn = out3d.shape
    grid = nb // rb
    res = pl.pallas_call(
        _tc_loss_block,
        grid=(grid,),
        in_specs=[
            pl.BlockSpec((t, rb, nn), lambda i: (0, i, 0)),
            pl.BlockSpec((rb, nn), lambda i: (i, 0)),
        ],
        out_specs=pl.BlockSpec((1, 128), lambda i: (0, 0)),
        out_shape=jax.ShapeDtypeStruct((1, 128), jnp.float32),
    )(out3d, tgt2d)
    return res[0, 0]


def kernel(output, target):
    t, nb, nn = output.shape
    part = _tc_partial(output, target.astype(jnp.int32), rb=256)
    return 0.5 * t * part
